# baseline (speedup 1.0000x reference)
import functools

import jax
import jax.numpy as jnp
from jax import lax
from jax.experimental import pallas as pl
from jax.experimental.pallas import tpu as pltpu

DIM = 768
DEPTH = 4
E = 2 ** DEPTH
HID = DIM * 4


def _step(x_ref, rw_ref, rb_ref, w1_ref, b1_ref, w2_ref, b2_ref, out_ref):
    e = pl.program_id(0)
    xv = x_ref[...]
    n = xv.shape[0]

    logits = lax.dot_general(
        xv, rw_ref[...], (((1,), (1,)), ((), ())),
        preferred_element_type=jnp.float32,
    ) + rb_ref[...]
    sig = jax.nn.sigmoid(logits)

    cols = []
    for leaf in range(E):
        f = None
        for d in range(DEPTH):
            node = (2 ** d - 1) + (leaf >> (DEPTH - d))
            bit = (leaf >> (DEPTH - 1 - d)) & 1
            s = sig[:, node:node + 1]
            term = s if bit else (1.0 - s)
            f = term if f is None else f * term
        cols.append(f)
    leaf_probs = jnp.concatenate(cols, axis=1)

    sel = lax.broadcasted_iota(jnp.int32, (n, E), 1) == e
    p_col = jnp.sum(jnp.where(sel, leaf_probs, 0.0), axis=1, keepdims=True)

    w1e = w1_ref[0]
    h = lax.dot_general(
        xv, w1e, (((1,), (1,)), ((), ())),
        preferred_element_type=jnp.float32,
    ) + b1_ref[0]
    h = h * 0.5 * (1.0 + lax.erf(h * 0.7071067811865476))
    w2e = w2_ref[0]
    o = lax.dot_general(
        p_col * h, w2e, (((1,), (1,)), ((), ())),
        preferred_element_type=jnp.float32,
    )
    o = o + p_col * b2_ref[0]

    @pl.when(e == 0)
    def _init():
        out_ref[...] = o

    @pl.when(e > 0)
    def _acc():
        out_ref[...] += o


@jax.jit
def _fastff(x, router_w, router_b, w1, b1, w2, b2):
    b_, s_, d_ = x.shape
    n = b_ * s_
    flat_x = x.reshape(n, d_)
    rw = jnp.concatenate([router_w, jnp.zeros((1, d_), router_w.dtype)], axis=0)
    rb = jnp.concatenate([router_b, jnp.zeros((1,), router_b.dtype)])[None, :]

    out = pl.pallas_call(
        _step,
        grid=(E,),
        in_specs=[
            pl.BlockSpec((n, DIM), lambda e: (0, 0)),
            pl.BlockSpec((E, DIM), lambda e: (0, 0)),
            pl.BlockSpec((1, E), lambda e: (0, 0)),
            pl.BlockSpec((1, HID, DIM), lambda e: (e, 0, 0)),
            pl.BlockSpec((1, 1, HID), lambda e: (e, 0, 0)),
            pl.BlockSpec((1, DIM, HID), lambda e: (e, 0, 0)),
            pl.BlockSpec((1, 1, DIM), lambda e: (e, 0, 0)),
        ],
        out_specs=pl.BlockSpec((n, DIM), lambda e: (0, 0)),
        out_shape=jax.ShapeDtypeStruct((n, DIM), x.dtype),
        compiler_params=pltpu.CompilerParams(
            dimension_semantics=("arbitrary",),
        ),
    )(flat_x, rw, rb, w1, b1[:, None, :], w2, b2[:, None, :])
    return out.reshape(b_, s_, d_)


def kernel(x, router_w, router_b, w1, b1, w2, b2):
    return _fastff(x, router_w, router_b, w1, b1, w2, b2)


# speedup vs baseline: 1.0197x; 1.0197x over previous
import functools

import jax
import jax.numpy as jnp
from jax import lax
from jax.experimental import pallas as pl
from jax.experimental.pallas import tpu as pltpu

DIM = 768
DEPTH = 4
E = 2 ** DEPTH
HID = DIM * 4
HBLK = 1536


def _step(x_ref, rw_ref, rb_ref, w1_ref, b1_ref, w2_ref, b2_ref, out_ref):
    e = pl.program_id(0)
    k = pl.program_id(1)
    xv = x_ref[...]
    n = xv.shape[0]

    logits = lax.dot_general(
        xv, rw_ref[...], (((1,), (1,)), ((), ())),
        preferred_element_type=jnp.float32,
    ) + rb_ref[...]
    sig = jax.nn.sigmoid(logits)

    cols = []
    for leaf in range(E):
        f = None
        for d in range(DEPTH):
            node = (2 ** d - 1) + (leaf >> (DEPTH - d))
            bit = (leaf >> (DEPTH - 1 - d)) & 1
            s = sig[:, node:node + 1]
            term = s if bit else (1.0 - s)
            f = term if f is None else f * term
        cols.append(f)
    leaf_probs = jnp.concatenate(cols, axis=1)

    sel = lax.broadcasted_iota(jnp.int32, (n, E), 1) == e
    p_col = jnp.sum(jnp.where(sel, leaf_probs, 0.0), axis=1, keepdims=True)

    w1e = w1_ref[0]
    h = lax.dot_general(
        xv, w1e, (((1,), (1,)), ((), ())),
        preferred_element_type=jnp.float32,
    ) + b1_ref[0]
    h = h * 0.5 * (1.0 + lax.erf(h * 0.7071067811865476))
    w2e = w2_ref[0]
    o = lax.dot_general(
        p_col * h, w2e, (((1,), (1,)), ((), ())),
        preferred_element_type=jnp.float32,
    )

    @pl.when(k == 0)
    def _bias():
        o2 = o + p_col * b2_ref[0]

        @pl.when(e == 0)
        def _init():
            out_ref[...] = o2

        @pl.when(e > 0)
        def _acc():
            out_ref[...] += o2

    @pl.when(k > 0)
    def _acc_k():
        out_ref[...] += o


@jax.jit
def _fastff(x, router_w, router_b, w1, b1, w2, b2):
    b_, s_, d_ = x.shape
    n = b_ * s_
    flat_x = x.reshape(n, d_)
    rw = jnp.concatenate([router_w, jnp.zeros((1, d_), router_w.dtype)], axis=0)
    rb = jnp.concatenate([router_b, jnp.zeros((1,), router_b.dtype)])[None, :]

    k_steps = HID // HBLK
    out = pl.pallas_call(
        _step,
        grid=(E, k_steps),
        in_specs=[
            pl.BlockSpec((n, DIM), lambda e, k: (0, 0)),
            pl.BlockSpec((E, DIM), lambda e, k: (0, 0)),
            pl.BlockSpec((1, E), lambda e, k: (0, 0)),
            pl.BlockSpec((1, HBLK, DIM), lambda e, k: (e, k, 0)),
            pl.BlockSpec((1, 1, HBLK), lambda e, k: (e, 0, k)),
            pl.BlockSpec((1, DIM, HBLK), lambda e, k: (e, 0, k)),
            pl.BlockSpec((1, 1, DIM), lambda e, k: (e, 0, 0)),
        ],
        out_specs=pl.BlockSpec((n, DIM), lambda e, k: (0, 0)),
        out_shape=jax.ShapeDtypeStruct((n, DIM), x.dtype),
        compiler_params=pltpu.CompilerParams(
            dimension_semantics=("arbitrary", "arbitrary"),
        ),
    )(flat_x, rw, rb, w1, b1[:, None, :], w2, b2[:, None, :])
    return out.reshape(b_, s_, d_)


def kernel(x, router_w, router_b, w1, b1, w2, b2):
    return _fastff(x, router_w, router_b, w1, b1, w2, b2)
